# Initial kernel scaffold; baseline (speedup 1.0000x reference)
import jax
import jax.numpy as jnp
from jax.experimental import pallas as pl


def kernel(x, edge_index, W_l1, b_l1, W_r1, W_l2, b_l2, W_r2):
    raise NotImplementedError("write your pallas kernel here")


# speedup vs baseline: 7.8883x; 7.8883x over previous
import functools

import jax
import jax.numpy as jnp
from jax import lax
from jax.experimental import pallas as pl
from jax.experimental.pallas import tpu as pltpu
from jax.experimental.pallas import tpu_sc as plsc

NC = 2
NS = 16
NW = NC * NS
B = 128


def _mm_body(x_ref, w_ref, o_ref):
    o_ref[...] = jnp.dot(x_ref[...], w_ref[...],
                         preferred_element_type=jnp.float32)


def _tc_matmul(x, w, blk):
    n, d = x.shape
    k = w.shape[1]
    grid = (n // blk,)
    return pl.pallas_call(
        _mm_body,
        grid=grid,
        in_specs=[pl.BlockSpec((blk, d), lambda i: (i, 0)),
                  pl.BlockSpec((d, k), lambda i: (0, 0))],
        out_specs=pl.BlockSpec((blk, k), lambda i: (i, 0)),
        out_shape=jax.ShapeDtypeStruct((n, k), jnp.float32),
    )(x, w)


def _mid_body(p0_ref, p1_ref, d0_ref, d1_ref, r1_ref, b1_ref, w_ref,
              y2_ref, r2_ref):
    deg = d0_ref[...][:, :1] + d1_ref[...][:, :1]
    rdeg = 1.0 / jnp.maximum(deg, 1.0)
    h = (p0_ref[...] + p1_ref[...]) * rdeg + b1_ref[...] + r1_ref[...]
    h = jnp.maximum(h, 0.0)
    yr = jnp.dot(h, w_ref[...], preferred_element_type=jnp.float32)
    dh = w_ref.shape[1] // 2
    y2_ref[...] = yr[:, :dh]
    r2_ref[...] = yr[:, dh:]


def _tc_mid(p0, p1, d0, d1, r1, b1, w, blk):
    n, dh = p0.shape
    do2 = w.shape[1]
    do = do2 // 2
    grid = (n // blk,)
    return pl.pallas_call(
        _mid_body,
        grid=grid,
        in_specs=[pl.BlockSpec((blk, dh), lambda i: (i, 0)),
                  pl.BlockSpec((blk, dh), lambda i: (i, 0)),
                  pl.BlockSpec((blk, 16), lambda i: (i, 0)),
                  pl.BlockSpec((blk, 16), lambda i: (i, 0)),
                  pl.BlockSpec((blk, dh), lambda i: (i, 0)),
                  pl.BlockSpec((1, dh), lambda i: (0, 0)),
                  pl.BlockSpec((dh, do2), lambda i: (0, 0))],
        out_specs=[pl.BlockSpec((blk, do), lambda i: (i, 0)),
                   pl.BlockSpec((blk, do), lambda i: (i, 0))],
        out_shape=[jax.ShapeDtypeStruct((n, do), jnp.float32),
                   jax.ShapeDtypeStruct((n, do), jnp.float32)],
    )(p0, p1, d0, d1, r1, b1, w)


def _fin_body(q0_ref, q1_ref, d0_ref, d1_ref, r2_ref, b2_ref, o_ref):
    deg = d0_ref[...][:, :1] + d1_ref[...][:, :1]
    rdeg = 1.0 / jnp.maximum(deg, 1.0)
    o_ref[...] = (q0_ref[...] + q1_ref[...]) * rdeg + b2_ref[...] + r2_ref[...]


def _tc_fin(q0, q1, d0, d1, r2, b2, blk):
    n, do = q0.shape
    grid = (n // blk,)
    return pl.pallas_call(
        _fin_body,
        grid=grid,
        in_specs=[pl.BlockSpec((blk, do), lambda i: (i, 0)),
                  pl.BlockSpec((blk, do), lambda i: (i, 0)),
                  pl.BlockSpec((blk, 16), lambda i: (i, 0)),
                  pl.BlockSpec((blk, 16), lambda i: (i, 0)),
                  pl.BlockSpec((blk, do), lambda i: (i, 0)),
                  pl.BlockSpec((1, do), lambda i: (0, 0))],
        out_specs=pl.BlockSpec((blk, do), lambda i: (i, 0)),
        out_shape=jax.ShapeDtypeStruct((n, do), jnp.float32),
    )(q0, q1, d0, d1, r2, b2)


def _sc_aggregate(table, srcr, dstr, ones16, zeros_d, zeros16, n_acc,
                  with_deg):
    d = table.shape[1]
    nb = srcr.shape[1]
    rpt = n_acc // NS
    nzc = rpt // B
    mesh = plsc.VectorSubcoreMesh(core_axis_name="c", subcore_axis_name="s")

    out_type = [jax.ShapeDtypeStruct((NC * n_acc, d), jnp.float32)]
    scratch = [
        pltpu.VMEM((nb, B), jnp.int32),
        pltpu.VMEM((nb, B), jnp.int32),
        pltpu.VMEM((B, d), jnp.float32),
        pltpu.VMEM_SHARED((n_acc, d), jnp.float32),
        pltpu.SemaphoreType.DMA,
    ]
    if with_deg:
        out_type.append(jax.ShapeDtypeStruct((NC * n_acc, 16), jnp.float32))
        scratch.insert(3, pltpu.VMEM((B, 16), jnp.float32))
        scratch.insert(5, pltpu.VMEM_SHARED((n_acc, 16), jnp.float32))

    def body(*refs):
        if with_deg:
            (tab, sr, dr, ones_h, zd_h, z16_h,
             pout, degout, src_v, dst_v, rows_v, ones_v,
             acc_sh, deg_sh, sem) = refs
        else:
            (tab, sr, dr, ones_h, zd_h, z16_h,
             pout, src_v, dst_v, rows_v, acc_sh, sem) = refs
            degout = deg_sh = ones_v = None
        c = lax.axis_index("c")
        s = lax.axis_index("s")
        wid = c * NS + s
        stripe = s * rpt

        pltpu.sync_copy(sr.at[wid], src_v)
        pltpu.sync_copy(dr.at[wid], dst_v)

        pltpu.sync_copy(zd_h, rows_v)
        for i in range(nzc):
            pltpu.sync_copy(rows_v, acc_sh.at[pl.ds(stripe + i * B, B)])
        if with_deg:
            pltpu.sync_copy(z16_h, ones_v)
            for i in range(nzc):
                pltpu.sync_copy(ones_v, deg_sh.at[pl.ds(stripe + i * B, B)])
            pltpu.sync_copy(ones_h, ones_v)
        plsc.subcore_barrier()

        def step(b, carry):
            pltpu.async_copy(tab.at[src_v.at[b]], rows_v, sem).wait()
            pltpu.sync_copy(rows_v, acc_sh.at[dst_v.at[b]], add=True)
            if with_deg:
                pltpu.sync_copy(ones_v, deg_sh.at[dst_v.at[b]], add=True)
            return carry

        lax.fori_loop(0, nb, step, 0)
        plsc.subcore_barrier()

        for i in range(nzc):
            r0 = stripe + i * B
            pltpu.sync_copy(acc_sh.at[pl.ds(r0, B)], rows_v)
            pltpu.sync_copy(rows_v, pout.at[pl.ds(c * n_acc + r0, B)])
            if with_deg:
                pltpu.sync_copy(deg_sh.at[pl.ds(r0, B)], ones_v)
                pltpu.sync_copy(ones_v, degout.at[pl.ds(c * n_acc + r0, B)])

    fn = pl.kernel(body, out_type=out_type, mesh=mesh, scratch_types=scratch,
                   compiler_params=pltpu.CompilerParams(
                       use_tc_tiling_on_sc=False))
    return fn(table, srcr, dstr, ones16, zeros_d, zeros16)


def kernel(x, edge_index, W_l1, b_l1, W_r1, W_l2, b_l2, W_r2):
    n, d_in = x.shape
    d_hid = W_l1.shape[1]
    d_out = W_l2.shape[1]
    e = edge_index.shape[1]

    blk = 1024
    n_acc = ((n + (NS * B) - 1) // (NS * B)) * (NS * B)
    e_tile = ((e + (NW * B) - 1) // (NW * B)) * B
    nb = e_tile // B
    e_pad = NW * e_tile

    src = edge_index[0].astype(jnp.int32)
    dst = edge_index[1].astype(jnp.int32)
    src = jnp.concatenate([src, jnp.zeros((e_pad - e,), jnp.int32)])
    dst = jnp.concatenate([dst, jnp.full((e_pad - e,), n, jnp.int32)])
    srcr = src.reshape(NW, nb, B)
    dstr = dst.reshape(NW, nb, B)

    ones16 = jnp.ones((B, 16), jnp.float32)
    zeros16 = jnp.zeros((B, 16), jnp.float32)
    zeros_hid = jnp.zeros((B, d_hid), jnp.float32)
    zeros_out = jnp.zeros((B, d_out), jnp.float32)

    x_pad = jnp.pad(x, ((0, n_acc - n), (0, 0)))

    wcat1 = jnp.concatenate([W_l1, W_r1], axis=1)
    yr1 = _tc_matmul(x_pad, wcat1, blk)
    y1 = yr1[:, :d_hid]
    r1 = yr1[:, d_hid:]

    pflat, degflat = _sc_aggregate(y1, srcr, dstr, ones16, zeros_hid,
                                   zeros16, n_acc, with_deg=True)
    p0 = pflat[:n_acc]
    p1 = pflat[n_acc:]
    d0 = degflat[:n_acc]
    d1 = degflat[n_acc:]

    wcat2 = jnp.concatenate([W_l2, W_r2], axis=1)
    y2, r2 = _tc_mid(p0, p1, d0, d1, r1, b_l1.reshape(1, d_hid), wcat2, blk)

    qflat = _sc_aggregate(y2, srcr, dstr, ones16, zeros_out, zeros16,
                          n_acc, with_deg=False)[0]
    q0 = qflat[:n_acc]
    q1 = qflat[n_acc:]

    out = _tc_fin(q0, q1, d0, d1, r2, b_l2.reshape(1, d_out), blk)
    return out[:n]
